# Initial kernel scaffold; baseline (speedup 1.0000x reference)
import jax
import jax.numpy as jnp
from jax.experimental import pallas as pl


def kernel(tokens, table):
    raise NotImplementedError("write your pallas kernel here")


# speedup vs baseline: 2.2498x; 2.2498x over previous
import functools
import math

import jax
import jax.numpy as jnp
from jax import lax
from jax.experimental import pallas as pl
from jax.experimental.pallas import tpu as pltpu
from jax.experimental.pallas import tpu_sc as plsc

_BATCH = 4096
_SEQ = 50
_B = _BATCH * _SEQ
_D = 128
_SCALE = math.sqrt(float(_D))

_NC, _NS = 2, 16
_NW = _NC * _NS
_BPW = _B // _NW
_G = 128
_NCH = _BPW // _G
_NBUF = 2

_mesh = plsc.VectorSubcoreMesh(core_axis_name="c", subcore_axis_name="s")


def _scale_chunk(src, dst):
    @pl.loop(0, _G, unroll=4)
    def _(r):
        for j in range(_D // 16):
            sl = pl.ds(j * 16, 16)
            dst[r, sl] = src[r, sl] * _SCALE


@functools.partial(
    pl.kernel,
    out_type=jax.ShapeDtypeStruct((_B, _D), jnp.float32),
    mesh=_mesh,
    scratch_types=[
        pltpu.VMEM((_BPW,), jnp.int32),
        pltpu.VMEM((_NBUF, _G, _D), jnp.float32),
        pltpu.VMEM((_NBUF, _G, _D), jnp.float32),
        pltpu.SemaphoreType.DMA,
        [pltpu.SemaphoreType.DMA] * _NBUF,
        [pltpu.SemaphoreType.DMA] * _NBUF,
    ],
)
def _emb_lookup(idx_hbm, table_hbm, out_hbm, idx_v, gbuf, sbuf,
                sem_i, sems_g, sems_s):
    wid = lax.axis_index("s") * _NC + lax.axis_index("c")
    base = wid * _BPW

    pltpu.async_copy(idx_hbm.at[pl.ds(base, _BPW)], idx_v, sem_i).wait()

    def issue_gather(g, b):
        pltpu.async_copy(table_hbm.at[idx_v.at[pl.ds(g * _G, _G)]],
                         gbuf.at[b], sems_g[b])

    def wait_gather(b):
        pltpu.make_async_copy(table_hbm.at[idx_v.at[pl.ds(0, _G)]],
                              gbuf.at[b], sems_g[b]).wait()

    def issue_scatter(g, b):
        pltpu.async_copy(sbuf.at[b], out_hbm.at[pl.ds(base + g * _G, _G)],
                         sems_s[b])

    def wait_scatter(b):
        pltpu.make_async_copy(sbuf.at[b], out_hbm.at[pl.ds(base, _G)],
                              sems_s[b]).wait()

    for b in range(_NBUF):
        issue_gather(b, b)

    @pl.loop(0, _NCH - _NBUF, step=_NBUF)
    def _(g0):
        for b in range(_NBUF):
            g = g0 + b
            wait_gather(b)
            @pl.when(g0 > 0)
            def _():
                wait_scatter(b)
            _scale_chunk(gbuf.at[b], sbuf.at[b])
            issue_gather(g + _NBUF, b)
            issue_scatter(g, b)

    for b in range(_NBUF):
        g = _NCH - _NBUF + b
        wait_gather(b)
        wait_scatter(b)
        _scale_chunk(gbuf.at[b], sbuf.at[b])
        issue_scatter(g, b)

    for b in range(_NBUF):
        wait_scatter(b)


def kernel(tokens, table):
    idx = tokens.astype(jnp.int32).reshape(_B)
    out = _emb_lookup(idx, table)
    return out.reshape(_BATCH, _SEQ, _D)
